# baseline (speedup 1.0000x reference)
import jax
import jax.numpy as jnp
import sc_kernel_mod


def kernel(input):
    b, c, h, w = input.shape
    n = b * h * w
    xt = jnp.transpose(input, (0, 2, 3, 1)).reshape(n, c)
    out_t = sc_kernel_mod.sc_shuffle(xt, n, c)
    return jnp.transpose(out_t.reshape(b, h, w, c), (0, 3, 1, 2))


# speedup vs baseline: 3.1068x; 3.1068x over previous
import numpy as np
import jax
import jax.numpy as jnp
from jax.experimental import pallas as pl

_GROUPS = 8


def _perm(channels, groups):
    cpg = channels // groups
    oc = np.arange(channels, dtype=np.int64)
    return oc // cpg + (oc % cpg) * groups


def kernel(input):
    b, c, h, w = input.shape
    n = b * h * w

    p = np.zeros((c, c), dtype=np.float32)
    p[_perm(c, _GROUPS), np.arange(c)] = 1.0
    p_arr = jnp.asarray(p)

    xt = jnp.transpose(input, (0, 2, 3, 1)).reshape(n, c)

    BR = 7168
    grid = (n // BR,)

    def body(x_ref, p_ref, o_ref):
        o_ref[...] = jax.lax.dot_general(
            x_ref[...], p_ref[...],
            dimension_numbers=(((1,), (0,)), ((), ())),
            preferred_element_type=jnp.float32,
        )

    out_t = pl.pallas_call(
        body,
        grid=grid,
        in_specs=[
            pl.BlockSpec((BR, c), lambda i: (i, 0)),
            pl.BlockSpec((c, c), lambda i: (0, 0)),
        ],
        out_specs=pl.BlockSpec((BR, c), lambda i: (i, 0)),
        out_shape=jax.ShapeDtypeStruct((n, c), jnp.float32),
    )(xt, p_arr)

    return jnp.transpose(out_t.reshape(b, h, w, c), (0, 3, 1, 2))
